# baseline (speedup 1.0000x reference)
import functools
import math

import jax
import jax.numpy as jnp
from jax import lax
from jax.experimental import pallas as pl
from jax.experimental.pallas import tpu as pltpu
from jax.experimental.pallas import tpu_sc as plsc

_TEMP = 0.05
_SHIFT = 20.0
_CHUNK = 512
_LOG2E = math.log2(math.e)
_BIAS_BF16 = float(jnp.float32(jnp.bfloat16(-_SHIFT * _LOG2E)))
_SHIFT_EXACT = -_BIAS_BF16 * math.log(2.0)


def _make_sc_gather(b, d, table_dtype):
    info = plsc.get_sparse_core_info()
    nc, ns = info.num_cores, info.num_subcores
    nw = nc * ns
    assert b % (8 * nw) == 0 and d % 128 == 0
    b_per_w = b // nw
    mesh = plsc.VectorSubcoreMesh(core_axis_name="c", subcore_axis_name="s")

    @functools.partial(
        pl.kernel, mesh=mesh,
        out_type=jax.ShapeDtypeStruct((b, d), table_dtype),
        scratch_types=[
            pltpu.VMEM((b_per_w,), jnp.int32),
            pltpu.VMEM((b_per_w, d), table_dtype),
            pltpu.SemaphoreType.DMA,
        ],
    )
    def gather_k(table_hbm, idx_hbm, out_hbm, idx_v, rows_v, sem):
        wid = lax.axis_index("s") * nc + lax.axis_index("c")
        base = wid * b_per_w
        pltpu.sync_copy(idx_hbm.at[pl.ds(base, b_per_w)], idx_v)
        pltpu.async_copy(table_hbm.at[idx_v], rows_v, sem).wait()
        pltpu.sync_copy(rows_v, out_hbm.at[pl.ds(base, b_per_w)])

    return gather_k


def _ce_kernel(n_valid, n_rows, x_ref, f_ref, g_ref, out_ref,
               xn_ref, s_ref):
    c = pl.program_id(0)
    nc = pl.num_programs(0)

    @pl.when(c == 0)
    def _init():
        x = x_ref[...]
        norm = jnp.sqrt(jnp.sum(x * x, axis=1, keepdims=True))
        scale = _LOG2E / (jnp.maximum(norm, 1e-12) * _TEMP)
        lane = jax.lax.broadcasted_iota(jnp.int32, x.shape, 1)
        onehot = jnp.where(lane == 64, 1.0, 0.0)
        xn_ref[...] = (x * scale + onehot).astype(jnp.bfloat16)
        s_ref[...] = jnp.zeros_like(s_ref)

    m = jax.lax.dot_general(
        xn_ref[...], f_ref[...], (((1,), (1,)), ((), ())),
        preferred_element_type=jnp.float32)
    ev = jnp.exp2(m)

    @pl.when(c < nc - 1)
    def _mid():
        s_ref[...] += ((ev[:, 0:128] + ev[:, 128:256])
                       + (ev[:, 256:384] + ev[:, 384:512]))

    @pl.when(c == nc - 1)
    def _tail():
        lane = jax.lax.broadcasted_iota(jnp.int32, ev.shape, 1)
        evm = jnp.where(lane < n_valid - c * _CHUNK, ev, 0.0)
        s_ref[...] += ((evm[:, 0:128] + evm[:, 128:256])
                       + (evm[:, 256:384] + evm[:, 384:512]))

    @pl.when(c == nc - 1)
    def _fin():
        lse = (jnp.log(jnp.sum(s_ref[...], axis=1, keepdims=True))
               + _SHIFT_EXACT)
        x = x_ref[...]
        norm = jnp.sqrt(jnp.sum(x * x, axis=1, keepdims=True))
        xh = x / (jnp.maximum(norm, 1e-12) * _TEMP)
        tl = jnp.sum(xh * g_ref[...])
        out_ref[...] = ((jnp.sum(lse) - tl) * (1.0 / n_rows)).reshape(1, 1)


@jax.jit
def kernel(inputs, targets, cameras, features):
    b, d = inputs.shape
    n = features.shape[0]
    nc = pl.cdiv(n, _CHUNK)
    n_pad = nc * _CHUNK
    fb = jnp.concatenate(
        [features.astype(jnp.bfloat16),
         jnp.full((n, 1), _BIAS_BF16, jnp.bfloat16),
         jnp.zeros((n, 127 - d), jnp.bfloat16)], axis=1)
    fb = jnp.pad(fb, ((0, n_pad - n), (0, 0)))
    xpad = jnp.pad(inputs, ((0, 0), (0, 128 - d)))
    f128 = jnp.pad(features, ((0, 0), (0, 128 - d)))
    g = _make_sc_gather(b, 128, jnp.float32)(f128, targets.astype(jnp.int32))
    out = pl.pallas_call(
        functools.partial(_ce_kernel, n, b),
        grid=(nc,),
        in_specs=[
            pl.BlockSpec((b, 128), lambda i: (0, 0)),
            pl.BlockSpec((_CHUNK, 128), lambda i: (i, 0)),
            pl.BlockSpec((b, 128), lambda i: (0, 0)),
        ],
        out_specs=pl.BlockSpec((1, 1), lambda i: (0, 0)),
        out_shape=jax.ShapeDtypeStruct((1, 1), jnp.float32),
        scratch_shapes=[
            pltpu.VMEM((b, 128), jnp.bfloat16),
            pltpu.VMEM((b, 128), jnp.float32),
        ],
        compiler_params=pltpu.CompilerParams(
            dimension_semantics=("arbitrary",)),
    )(xpad, fb, g)
    return out[0, 0]


# speedup vs baseline: 1.0575x; 1.0575x over previous
import functools
import math

import jax
import jax.numpy as jnp
from jax import lax
from jax.experimental import pallas as pl
from jax.experimental.pallas import tpu as pltpu
from jax.experimental.pallas import tpu_sc as plsc

_TEMP = 0.05
_CHUNK = 512
_LOG2E = math.log2(math.e)


def _make_sc_gather(b, d):
    info = plsc.get_sparse_core_info()
    nc, ns = info.num_cores, info.num_subcores
    nw = nc * ns
    assert b % (8 * nw) == 0 and d % 128 == 0
    b_per_w = b // nw
    mesh = plsc.VectorSubcoreMesh(core_axis_name="c", subcore_axis_name="s")

    @functools.partial(
        pl.kernel, mesh=mesh,
        out_type=jax.ShapeDtypeStruct((b, d), jnp.float32),
        scratch_types=[
            pltpu.VMEM((b_per_w,), jnp.int32),
            pltpu.VMEM((b_per_w, d), jnp.float32),
            pltpu.SemaphoreType.DMA,
        ],
    )
    def gather_k(table_hbm, idx_hbm, out_hbm, idx_v, rows_v, sem):
        wid = lax.axis_index("s") * nc + lax.axis_index("c")
        base = wid * b_per_w
        pltpu.sync_copy(idx_hbm.at[pl.ds(base, b_per_w)], idx_v)
        pltpu.async_copy(table_hbm.at[idx_v], rows_v, sem).wait()
        pltpu.sync_copy(rows_v, out_hbm.at[pl.ds(base, b_per_w)])

    return gather_k


def _ce_kernel(n_valid, n_rows, x_ref, f_ref, g_ref, out_ref,
               xn_ref, s_ref):
    c = pl.program_id(0)
    nc = pl.num_programs(0)

    @pl.when(c == 0)
    def _init():
        x = x_ref[...]
        norm = jnp.sqrt(jnp.sum(x * x, axis=1, keepdims=True))
        scale = _LOG2E / (jnp.maximum(norm, 1e-12) * _TEMP)
        xn_ref[...] = (x * scale).astype(jnp.bfloat16)
        s_ref[...] = jnp.zeros_like(s_ref)

    m = jax.lax.dot_general(
        xn_ref[...], f_ref[...], (((1,), (1,)), ((), ())),
        preferred_element_type=jnp.float32)
    ev = jnp.exp2(m)

    @pl.when(c < nc - 1)
    def _mid():
        s_ref[...] += ((ev[:, 0:128] + ev[:, 128:256])
                       + (ev[:, 256:384] + ev[:, 384:512]))

    @pl.when(c == nc - 1)
    def _tail():
        lane = jax.lax.broadcasted_iota(jnp.int32, ev.shape, 1)
        evm = jnp.where(lane < n_valid - c * _CHUNK, ev, 0.0)
        s_ref[...] += ((evm[:, 0:128] + evm[:, 128:256])
                       + (evm[:, 256:384] + evm[:, 384:512]))

    @pl.when(c == nc - 1)
    def _fin():
        lse = jnp.log(jnp.sum(s_ref[...], axis=1, keepdims=True))
        x = x_ref[...]
        norm = jnp.sqrt(jnp.sum(x * x, axis=1, keepdims=True))
        xh = x / (jnp.maximum(norm, 1e-12) * _TEMP)
        tl = jnp.sum(xh * g_ref[:, 0:64])
        out_ref[...] = ((jnp.sum(lse) - tl) * (1.0 / n_rows)).reshape(1, 1)


@jax.jit
def kernel(inputs, targets, cameras, features):
    b, d = inputs.shape
    n = features.shape[0]
    nc = pl.cdiv(n, _CHUNK)
    n_pad = nc * _CHUNK
    fpad = jnp.pad(features.astype(jnp.bfloat16), ((0, n_pad - n), (0, 0)))
    f128 = jnp.pad(features, ((0, 0), (0, 128 - d)))
    g = _make_sc_gather(b, 128)(f128, targets.astype(jnp.int32))
    out = pl.pallas_call(
        functools.partial(_ce_kernel, n, b),
        grid=(nc,),
        in_specs=[
            pl.BlockSpec((b, d), lambda i: (0, 0)),
            pl.BlockSpec((_CHUNK, d), lambda i: (i, 0)),
            pl.BlockSpec((b, 128), lambda i: (0, 0)),
        ],
        out_specs=pl.BlockSpec((1, 1), lambda i: (0, 0)),
        out_shape=jax.ShapeDtypeStruct((1, 1), jnp.float32),
        scratch_shapes=[
            pltpu.VMEM((b, d), jnp.bfloat16),
            pltpu.VMEM((b, 128), jnp.float32),
        ],
        compiler_params=pltpu.CompilerParams(
            dimension_semantics=("arbitrary",)),
    )(inputs, fpad, g)
    return out[0, 0]
